# baseline (speedup 1.0000x reference)
import functools

import jax
import jax.numpy as jnp
from jax import lax
from jax.experimental import pallas as pl
from jax.experimental.pallas import tpu as pltpu
from jax.experimental.pallas import tpu_sc as plsc


def kernel(categorical_features, emb_tables):
    B, F = categorical_features.shape
    Ft, V, D = emb_tables.shape
    assert Ft == F

    tabT = emb_tables.transpose(2, 0, 1)
    idxT = categorical_features.astype(jnp.int32).T

    info = plsc.get_sparse_core_info()
    NC, NS, L = info.num_cores, info.num_subcores, info.num_lanes
    NW = NC * NS
    NT = F * D
    BH = 4096
    NH = B // BH
    assert B % BH == 0 and BH % L == 0 and NH >= 2

    mesh = plsc.VectorSubcoreMesh(core_axis_name="c", subcore_axis_name="s")

    @functools.partial(
        pl.kernel,
        mesh=mesh,
        out_type=jax.ShapeDtypeStruct((NT, B), jnp.float32),
        scratch_types=[
            pltpu.VMEM((B,), jnp.int32),
            pltpu.VMEM((V,), jnp.float32),
            pltpu.VMEM((BH,), jnp.float32),
            pltpu.VMEM((BH,), jnp.float32),
            pltpu.SemaphoreType.DMA,
            pltpu.SemaphoreType.DMA,
        ],
        compiler_params=pltpu.CompilerParams(needs_layout_passes=False),
    )
    def sc_lookup(
        idx_hbm, tab_hbm, out_hbm, idx_v, row_v, out_v0, out_v1, sem_t, sem_o
    ):
        wid = lax.axis_index("s") * NC + lax.axis_index("c")
        lo = (wid * NT) // NW
        hi = ((wid + 1) * NT) // NW

        @pl.when(lax.axis_index("s") % 2 == 1)
        def _():
            pl.delay(3000)

        def task(t, prev_f):
            f = t // D
            d = t - f * D
            lane_cp = pltpu.async_copy(tab_hbm.at[d, f], row_v, sem_t)

            @pl.when(f != prev_f)
            def _():
                pltpu.sync_copy(idx_hbm.at[f], idx_v)

            lane_cp.wait()

            for h in range(NH):
                buf = out_v0 if h % 2 == 0 else out_v1
                g = (t - lo) * NH + h

                @pl.when(g >= 2)
                def _():
                    pltpu.make_async_copy(
                        buf, out_hbm.at[t, pl.ds(h * BH, BH)], sem_o
                    ).wait()

                @plsc.parallel_loop(0, BH // L, unroll=16)
                def gather_vec(j):
                    vec_idx = idx_v[pl.ds(h * BH + j * L, L)]
                    buf[pl.ds(j * L, L)] = plsc.load_gather(row_v, [vec_idx])

                pltpu.async_copy(buf, out_hbm.at[t, pl.ds(h * BH, BH)], sem_o)
            return f

        lax.fori_loop(lo, hi, task, -1)

        @pl.when(hi > lo)
        def _():
            for _ in range(2):
                pltpu.make_async_copy(
                    out_v0, out_hbm.at[0, pl.ds(0, BH)], sem_o
                ).wait()

    out = sc_lookup(idxT, tabT)
    return out.T


# speedup vs baseline: 1.0643x; 1.0643x over previous
import functools

import jax
import jax.numpy as jnp
from jax import lax
from jax.experimental import pallas as pl
from jax.experimental.pallas import tpu as pltpu
from jax.experimental.pallas import tpu_sc as plsc


def kernel(categorical_features, emb_tables):
    B, F = categorical_features.shape
    Ft, V, D = emb_tables.shape
    assert Ft == F

    tabT = emb_tables.transpose(2, 0, 1)
    idxT = categorical_features.astype(jnp.int32).T

    info = plsc.get_sparse_core_info()
    NC, NS, L = info.num_cores, info.num_subcores, info.num_lanes
    NW = NC * NS
    NT = F * D
    BH = 4096
    NH = B // BH
    assert B % BH == 0 and BH % L == 0 and NH >= 2

    mesh = plsc.VectorSubcoreMesh(core_axis_name="c", subcore_axis_name="s")

    @functools.partial(
        pl.kernel,
        mesh=mesh,
        out_type=jax.ShapeDtypeStruct((NT, B), jnp.float32),
        scratch_types=[
            pltpu.VMEM((B,), jnp.int32),
            pltpu.VMEM((V,), jnp.float32),
            pltpu.VMEM((BH,), jnp.float32),
            pltpu.VMEM((BH,), jnp.float32),
            pltpu.SemaphoreType.DMA,
            pltpu.SemaphoreType.DMA,
        ],
        compiler_params=pltpu.CompilerParams(needs_layout_passes=False),
    )
    def sc_lookup(
        idx_hbm, tab_hbm, out_hbm, idx_v, row_v, out_v0, out_v1, sem_t, sem_o
    ):
        wid = lax.axis_index("s") * NC + lax.axis_index("c")
        lo = (wid * NT) // NW
        hi = ((wid + 1) * NT) // NW

        def task(t, prev_f):
            f = t // D
            d = t - f * D
            lane_cp = pltpu.async_copy(tab_hbm.at[d, f], row_v, sem_t)

            @pl.when(f != prev_f)
            def _():
                pltpu.sync_copy(idx_hbm.at[f], idx_v)

            lane_cp.wait()

            for h in range(NH):
                buf = out_v0 if h % 2 == 0 else out_v1
                g = (t - lo) * NH + h

                @pl.when(g >= 2)
                def _():
                    pltpu.make_async_copy(
                        buf, out_hbm.at[t, pl.ds(h * BH, BH)], sem_o
                    ).wait()

                @plsc.parallel_loop(0, BH // L, unroll=16)
                def gather_vec(j):
                    vec_idx = idx_v[pl.ds(h * BH + j * L, L)]
                    buf[pl.ds(j * L, L)] = plsc.load_gather(row_v, [vec_idx])

                pltpu.async_copy(buf, out_hbm.at[t, pl.ds(h * BH, BH)], sem_o)
            return f

        lax.fori_loop(lo, hi, task, -1)

        @pl.when(hi > lo)
        def _():
            for _ in range(2):
                pltpu.make_async_copy(
                    out_v0, out_hbm.at[0, pl.ds(0, BH)], sem_o
                ).wait()

    out = sc_lookup(idxT, tabT)
    return out.T


# speedup vs baseline: 1.0661x; 1.0016x over previous
import functools

import jax
import jax.numpy as jnp
from jax import lax
from jax.experimental import pallas as pl
from jax.experimental.pallas import tpu as pltpu
from jax.experimental.pallas import tpu_sc as plsc


def kernel(categorical_features, emb_tables):
    B, F = categorical_features.shape
    Ft, V, D = emb_tables.shape
    assert Ft == F

    tabT = emb_tables.transpose(2, 0, 1)
    idxT = categorical_features.astype(jnp.int32).T

    info = plsc.get_sparse_core_info()
    NC, NS, L = info.num_cores, info.num_subcores, info.num_lanes
    NW = NC * NS
    NT = F * D
    BH = 4096
    NH = B // BH
    assert B % BH == 0 and BH % L == 0 and NH >= 2

    mesh = plsc.VectorSubcoreMesh(core_axis_name="c", subcore_axis_name="s")

    @functools.partial(
        pl.kernel,
        mesh=mesh,
        out_type=jax.ShapeDtypeStruct((NT, B), jnp.float32),
        scratch_types=[
            pltpu.VMEM((B,), jnp.int32),
            pltpu.VMEM((V,), jnp.float32),
            pltpu.VMEM((BH,), jnp.float32),
            pltpu.VMEM((BH,), jnp.float32),
            pltpu.SemaphoreType.DMA,
            pltpu.SemaphoreType.DMA,
            pltpu.SemaphoreType.DMA,
        ],
        compiler_params=pltpu.CompilerParams(needs_layout_passes=False),
    )
    def sc_lookup(
        idx_hbm, tab_hbm, out_hbm, idx_v, row_v, out_v0, out_v1,
        sem_t, sem_o0, sem_o1,
    ):
        wid = lax.axis_index("s") * NC + lax.axis_index("c")
        lo = (wid * NT) // NW
        hi = ((wid + 1) * NT) // NW

        def task(t, prev_f):
            f = t // D
            d = t - f * D
            lane_cp = pltpu.async_copy(tab_hbm.at[d, f], row_v, sem_t)

            @pl.when(f != prev_f)
            def _():
                pltpu.sync_copy(idx_hbm.at[f], idx_v)

            lane_cp.wait()

            for h in range(NH):
                buf = out_v0 if h % 2 == 0 else out_v1
                sem = sem_o0 if h % 2 == 0 else sem_o1
                g = (t - lo) * NH + h

                @pl.when(g >= 2)
                def _():
                    pltpu.make_async_copy(
                        buf, out_hbm.at[t, pl.ds(h * BH, BH)], sem
                    ).wait()

                @plsc.parallel_loop(0, BH // L, unroll=16)
                def gather_vec(j):
                    vec_idx = idx_v[pl.ds(h * BH + j * L, L)]
                    buf[pl.ds(j * L, L)] = plsc.load_gather(row_v, [vec_idx])

                pltpu.async_copy(buf, out_hbm.at[t, pl.ds(h * BH, BH)], sem)
            return f

        lax.fori_loop(lo, hi, task, -1)

        @pl.when(hi > lo)
        def _():
            pltpu.make_async_copy(
                out_v0, out_hbm.at[0, pl.ds(0, BH)], sem_o0
            ).wait()
            pltpu.make_async_copy(
                out_v1, out_hbm.at[0, pl.ds(0, BH)], sem_o1
            ).wait()

    out = sc_lookup(idxT, tabT)
    return out.T
